# Initial kernel scaffold; baseline (speedup 1.0000x reference)
import jax
import jax.numpy as jnp
from jax.experimental import pallas as pl


def kernel(x, weight):
    raise NotImplementedError("write your pallas kernel here")


# speedup vs baseline: 1.6835x; 1.6835x over previous
import functools

import jax
import jax.numpy as jnp
from jax import lax
from jax.experimental import pallas as pl
from jax.experimental.pallas import tpu as pltpu
from jax.experimental.pallas import tpu_sc as plsc

NC = 2
NS = 16
NW = NC * NS
M = 128


@functools.partial(jax.jit, static_argnames=("n_steps",))
def _sc_gather(weight, idx, n_steps):
    V, D = weight.shape
    B = NW * n_steps * M
    mesh = plsc.VectorSubcoreMesh(core_axis_name="c", subcore_axis_name="s")

    @functools.partial(
        pl.kernel,
        out_type=jax.ShapeDtypeStruct((B, D), jnp.float32),
        mesh=mesh,
        scratch_types=[
            pltpu.VMEM((n_steps, M), jnp.int32),
            pltpu.VMEM((M, D), jnp.float32),
            pltpu.SemaphoreType.DMA,
        ],
        compiler_params=pltpu.CompilerParams(use_tc_tiling_on_sc=False),
    )
    def k(table_hbm, idx_hbm, out_hbm, idx_v, rows_v, gsem):
        wid = lax.axis_index("s") * NC + lax.axis_index("c")
        pltpu.sync_copy(idx_hbm.at[wid], idx_v)
        base = wid * (n_steps * M)

        @pl.loop(0, n_steps)
        def _(j):
            pltpu.async_copy(table_hbm.at[idx_v.at[j]], rows_v, gsem).wait()
            pltpu.sync_copy(rows_v, out_hbm.at[pl.ds(base + j * M, M)])

    return k(weight, idx)


def kernel(x, weight):
    B0, B1 = x.shape
    V, D = weight.shape
    B = B0 * B1
    n_steps = B // (NW * M)
    idx = x.reshape(B).astype(jnp.int32).reshape(NW, n_steps, M)
    out = _sc_gather(weight, idx, n_steps)
    return out.reshape(B0, B1, D)


# speedup vs baseline: 1.8795x; 1.1164x over previous
import functools

import jax
import jax.numpy as jnp
from jax import lax
from jax.experimental import pallas as pl
from jax.experimental.pallas import tpu as pltpu
from jax.experimental.pallas import tpu_sc as plsc

NC = 2
NS = 16
NW = NC * NS
M = 128


@functools.partial(jax.jit, static_argnames=("n_steps",))
def _sc_gather(weight, idx, n_steps):
    V, D = weight.shape
    B = NW * n_steps * M
    mesh = plsc.VectorSubcoreMesh(core_axis_name="c", subcore_axis_name="s")

    NBUF = 4
    assert n_steps % NBUF == 0 and n_steps > NBUF

    @functools.partial(
        pl.kernel,
        out_type=jax.ShapeDtypeStruct((B, D), jnp.float32),
        mesh=mesh,
        scratch_types=[
            pltpu.VMEM((n_steps, M), jnp.int32),
            [pltpu.VMEM((M, D), jnp.float32) for _ in range(NBUF)],
            pltpu.SemaphoreType.DMA,
        ],
        compiler_params=pltpu.CompilerParams(use_tc_tiling_on_sc=False),
    )
    def k(table_hbm, idx_hbm, out_hbm, idx_v, rows, gsem):
        wid = lax.axis_index("s") * NC + lax.axis_index("c")
        pltpu.sync_copy(idx_hbm.at[wid], idx_v)
        base = wid * (n_steps * M)

        for b in range(NBUF):
            pltpu.async_copy(table_hbm.at[idx_v.at[b]], rows[b], gsem)

        @pl.loop(0, n_steps - NBUF, step=NBUF)
        def _(i):
            for b in range(NBUF):
                j = i + b
                pltpu.make_async_copy(table_hbm.at[pl.ds(0, M)], rows[b], gsem).wait()
                pltpu.sync_copy(rows[b], out_hbm.at[pl.ds(base + j * M, M)])
                pltpu.async_copy(table_hbm.at[idx_v.at[j + NBUF]], rows[b], gsem)

        for b in range(NBUF):
            j = n_steps - NBUF + b
            pltpu.make_async_copy(table_hbm.at[pl.ds(0, M)], rows[b], gsem).wait()
            pltpu.sync_copy(rows[b], out_hbm.at[pl.ds(base + j * M, M)])

    return k(weight, idx)


def kernel(x, weight):
    B0, B1 = x.shape
    V, D = weight.shape
    B = B0 * B1
    n_steps = B // (NW * M)
    idx = x.reshape(B).astype(jnp.int32).reshape(NW, n_steps, M)
    out = _sc_gather(weight, idx, n_steps)
    return out.reshape(B0, B1, D)
